# baseline (speedup 1.0000x reference)
import functools

import jax
import jax.numpy as jnp
from jax import lax
from jax.experimental import pallas as pl
from jax.experimental.pallas import tpu as pltpu
from jax.experimental.pallas import tpu_sc as plsc

_LANES = 16


def _node_prep_body(x_ref, w_ref, b_ref, p_ref, q_ref):
    xw = jnp.dot(x_ref[...], w_ref[...], preferred_element_type=jnp.float32)
    p_ref[...] = xw[:, :_LANES]
    q_ref[...] = xw[:, _LANES:] + b_ref[...]


def _node_prep(x, w_big, b16, block):
    n, d = x.shape
    return pl.pallas_call(
        _node_prep_body,
        grid=(n // block,),
        in_specs=[
            pl.BlockSpec((block, d), lambda i: (i, 0)),
            pl.BlockSpec(w_big.shape, lambda i: (0, 0)),
            pl.BlockSpec(b16.shape, lambda i: (0, 0)),
        ],
        out_specs=[
            pl.BlockSpec((block, _LANES), lambda i: (i, 0)),
            pl.BlockSpec((block, _LANES), lambda i: (i, 0)),
        ],
        out_shape=[
            jax.ShapeDtypeStruct((n, _LANES), jnp.float32),
            jax.ShapeDtypeStruct((n, _LANES), jnp.float32),
        ],
    )(x, w_big, b16)


def _epilogue_body(a0_ref, a1_ref, p_ref, w2_ref, b2_ref, o_ref):
    a = jnp.maximum(a0_ref[...][:, :10], a1_ref[...][:, :10])
    p = p_ref[...][:, :10]
    m = jnp.where(jnp.isneginf(a), 0.0, a + p)
    o_ref[...] = jnp.dot(m, w2_ref[...], preferred_element_type=jnp.float32) + b2_ref[...]


def _epilogue(acc2, p16, w2, b2row, block):
    n = p16.shape[0]
    nblk = n // block
    d_out = w2.shape[1]
    return pl.pallas_call(
        _epilogue_body,
        grid=(nblk,),
        in_specs=[
            pl.BlockSpec((block, _LANES), lambda i: (i, 0)),
            pl.BlockSpec((block, _LANES), lambda i: (i + nblk, 0)),
            pl.BlockSpec((block, _LANES), lambda i: (i, 0)),
            pl.BlockSpec(w2.shape, lambda i: (0, 0)),
            pl.BlockSpec(b2row.shape, lambda i: (0, 0)),
        ],
        out_specs=pl.BlockSpec((block, d_out), lambda i: (i, 0)),
        out_shape=jax.ShapeDtypeStruct((n, d_out), jnp.float32),
    )(acc2, acc2, p16, w2, b2row)


def _make_sc_aggregate(n, e):
    info = plsc.get_sparse_core_info()
    nc, ns, lanes = info.num_cores, info.num_subcores, info.num_lanes
    assert lanes == _LANES and n % ns == 0 and e % nc == 0
    npw = n // ns
    half = e // nc
    ch = 6400
    unroll = 4
    assert half % ch == 0 and (ch // _LANES) % unroll == 0
    nchunk = half // ch
    gb = 512

    mesh = plsc.VectorSubcoreMesh(core_axis_name="c", subcore_axis_name="s")

    @functools.partial(
        pl.kernel,
        mesh=mesh,
        compiler_params=pltpu.CompilerParams(
            needs_layout_passes=False, use_tc_tiling_on_sc=False
        ),
        out_type=jax.ShapeDtypeStruct((nc * n * _LANES,), jnp.float32),
        scratch_types=[
            pltpu.VMEM((npw * _LANES,), jnp.float32),
            pltpu.VMEM((ch,), jnp.int32),
            pltpu.VMEM((ch + _LANES,), jnp.int32),
            pltpu.VMEM((ch + _LANES,), jnp.int32),
            pltpu.VMEM((gb, _LANES), jnp.float32),
            pltpu.VMEM((gb,), jnp.float32),
            pltpu.VMEM((gb,), jnp.float32),
            pltpu.VMEM((gb,), jnp.int32),
            pltpu.VMEM((gb,), jnp.int32),
            pltpu.VMEM((gb,), jnp.int32),
            pltpu.VMEM((2, _LANES), jnp.float32),
            pltpu.SemaphoreType.DMA,
        ],
    )
    def sc_agg(dst_hbm, src_hbm, distf_hbm, q_hbm, c_hbm, acc_hbm,
               acc_v, dst_v, cid_v, dlc_v, qrow_v, d0_v, d1_v, i0_v, i1_v,
               sg_v, cw_v, sem):
        core = lax.axis_index("c")
        sid = lax.axis_index("s")
        node_lo = sid * npw
        iota = lax.iota(jnp.int32, _LANES)
        ninf = jnp.full((_LANES,), -jnp.inf, jnp.float32)

        pltpu.sync_copy(c_hbm, cw_v)

        def init_body(r, _):
            plsc.store_scatter(acc_v, [r * _LANES + iota], ninf)
            return 0
        lax.fori_loop(0, npw, init_body, 0)

        def cinit_body(r, _):
            plsc.store_scatter(cid_v, [r * _LANES + iota],
                               jnp.full((_LANES,), sid, jnp.int32))
            return 0
        lax.fori_loop(0, (ch + _LANES) // _LANES, cinit_body, 0)

        def accum_batch(b, k):
            cids = cid_v.at[pl.ds(b * gb, gb)]
            for g in range(gb // _LANES):
                idv = plsc.load_gather(cid_v, [b * gb + g * _LANES + iota])
                i0_v[g * _LANES:(g + 1) * _LANES] = idv * 2
                i1_v[g * _LANES:(g + 1) * _LANES] = idv * 2 + 1
            cp_0 = pltpu.async_copy(distf_hbm.at[i0_v], d0_v, sem)
            cp_1 = pltpu.async_copy(distf_hbm.at[i1_v], d1_v, sem)
            cp_s = pltpu.async_copy(src_hbm.at[cids], sg_v, sem)
            cp_0.wait()
            cp_1.wait()
            cp_s.wait()
            pltpu.async_copy(q_hbm.at[sg_v], qrow_v, sem).wait()

            c0 = cw_v[0, :]
            c1 = cw_v[1, :]
            valid = k - b * gb

            def edge_body(j, _):
                rows = jnp.full((_LANES,), j, jnp.int32)
                dvec = plsc.load_gather(dlc_v, [jnp.full((_LANES,), b * gb + j,
                                                         jnp.int32)])
                d0 = plsc.load_gather(d0_v, [rows])
                d1 = plsc.load_gather(d1_v, [rows])
                qrow = plsc.load_gather(qrow_v, [rows, iota])
                mrow = d0 * c0 + d1 * c1 + qrow
                aidx = dvec * _LANES + iota
                arow = plsc.load_gather(acc_v, [aidx])
                plsc.store_scatter(acc_v, [aidx], jnp.maximum(arow, mrow))
                return 0
            lax.fori_loop(0, jnp.minimum(valid, gb), edge_body, 0)
            return k

        def chunk_body(c, _):
            base = core * half + c * ch
            pltpu.sync_copy(dst_hbm.at[pl.ds(base, ch)], dst_v)

            def scan_body(t, cur):
                i0 = t * unroll
                masks, cnts, idsl, dus = [], [], [], []
                for u in range(unroll):
                    dvec = plsc.load_gather(dst_v, [(i0 + u) * _LANES + iota])
                    du = dvec - node_lo
                    m = (du >= 0) & (du < npw)
                    masks.append(m)
                    dus.append(du)
                    idsl.append(base + (i0 + u) * _LANES + iota)
                    cnts.append(jnp.sum(m.astype(jnp.int32)))
                cc = cur
                for u in range(unroll):
                    plsc.store_compressed(
                        cid_v.at[pl.ds(cc, _LANES)], idsl[u], mask=masks[u])
                    plsc.store_compressed(
                        dlc_v.at[pl.ds(cc, _LANES)], dus[u], mask=masks[u])
                    cc = cc + cnts[u]
                return cc

            k = lax.fori_loop(0, ch // _LANES // unroll, scan_body, 0)
            nb = (k + gb - 1) // gb
            lax.fori_loop(0, nb, accum_batch, k)
            return 0

        lax.fori_loop(0, nchunk, chunk_body, 0)

        out_off = (core * n + node_lo) * _LANES
        pltpu.sync_copy(acc_v, acc_hbm.at[pl.ds(out_off, npw * _LANES)])

    return sc_agg


def kernel(x, edge_index, dist, W1, b1, W2, b2):
    n, d_feat = x.shape
    e = edge_index.shape[1]
    d_hid = W1.shape[1]
    src = edge_index[0]
    dst = edge_index[1]
    A = W1[0:d_feat]
    B = W1[d_feat:2 * d_feat]
    C = W1[2 * d_feat:]

    zpad = jnp.zeros((d_feat, _LANES - d_hid), jnp.float32)
    w_big = jnp.concatenate([A - B, zpad, B, zpad], axis=1)
    b16 = jnp.concatenate([b1, jnp.zeros((_LANES - d_hid,), jnp.float32)]).reshape(1, _LANES)
    p16, q16 = _node_prep(x, w_big, b16, block=4000)

    c16 = jnp.concatenate([C, jnp.zeros((2, _LANES - d_hid), jnp.float32)], axis=1)
    accf = _make_sc_aggregate(n, e)(dst, src, dist.reshape(-1), q16, c16)
    acc2 = accf.reshape(2 * n, _LANES)

    return _epilogue(acc2, p16, W2, b2.reshape(1, -1), block=4000)


# speedup vs baseline: 1.1391x; 1.1391x over previous
import functools

import jax
import jax.numpy as jnp
from jax import lax
from jax.experimental import pallas as pl
from jax.experimental.pallas import tpu as pltpu
from jax.experimental.pallas import tpu_sc as plsc

_LANES = 16


def _node_prep_body(x_ref, w_ref, p_ref, q_ref):
    xw = jnp.dot(x_ref[...], w_ref[...], preferred_element_type=jnp.float32)
    p_ref[...] = xw[:, :_LANES]
    q_ref[...] = xw[:, _LANES:]


def _node_prep(x, w_big, block):
    n, d = x.shape
    return pl.pallas_call(
        _node_prep_body,
        grid=(n // block,),
        in_specs=[
            pl.BlockSpec((block, d), lambda i: (i, 0)),
            pl.BlockSpec(w_big.shape, lambda i: (0, 0)),
        ],
        out_specs=[
            pl.BlockSpec((block, _LANES), lambda i: (i, 0)),
            pl.BlockSpec((block, _LANES), lambda i: (i, 0)),
        ],
        out_shape=[
            jax.ShapeDtypeStruct((n, _LANES), jnp.float32),
            jax.ShapeDtypeStruct((n, _LANES), jnp.float32),
        ],
    )(x, w_big)


def _edge_prep_body(dist_ref, srcf_ref, dstf_ref, c_ref, b_ref, rec_ref):
    r = jnp.dot(dist_ref[...], c_ref[...], preferred_element_type=jnp.float32)
    r = r + b_ref[...]
    rec_ref[...] = jnp.concatenate(
        [
            r[:, :10],
            srcf_ref[...],
            dstf_ref[...],
            jnp.zeros((r.shape[0], 4), jnp.float32),
        ],
        axis=1,
    )


def _edge_prep(dist, srcf, dstf, c16, b16, block):
    e = dist.shape[0]
    return pl.pallas_call(
        _edge_prep_body,
        grid=(e // block,),
        in_specs=[
            pl.BlockSpec((block, 2), lambda i: (i, 0)),
            pl.BlockSpec((block, 1), lambda i: (i, 0)),
            pl.BlockSpec((block, 1), lambda i: (i, 0)),
            pl.BlockSpec(c16.shape, lambda i: (0, 0)),
            pl.BlockSpec(b16.shape, lambda i: (0, 0)),
        ],
        out_specs=pl.BlockSpec((block, _LANES), lambda i: (i, 0)),
        out_shape=jax.ShapeDtypeStruct((e, _LANES), jnp.float32),
    )(dist, srcf, dstf, c16, b16)


def _epilogue_body(a0_ref, a1_ref, p_ref, w2_ref, b2_ref, o_ref):
    a = jnp.maximum(a0_ref[...][:, :10], a1_ref[...][:, :10])
    p = p_ref[...][:, :10]
    m = jnp.where(jnp.isneginf(a), 0.0, a + p)
    o_ref[...] = jnp.dot(m, w2_ref[...], preferred_element_type=jnp.float32) + b2_ref[...]


def _epilogue(acc2, p16, w2, b2row, block):
    n = p16.shape[0]
    nblk = n // block
    d_out = w2.shape[1]
    return pl.pallas_call(
        _epilogue_body,
        grid=(nblk,),
        in_specs=[
            pl.BlockSpec((block, _LANES), lambda i: (i, 0)),
            pl.BlockSpec((block, _LANES), lambda i: (i + nblk, 0)),
            pl.BlockSpec((block, _LANES), lambda i: (i, 0)),
            pl.BlockSpec(w2.shape, lambda i: (0, 0)),
            pl.BlockSpec(b2row.shape, lambda i: (0, 0)),
        ],
        out_specs=pl.BlockSpec((block, d_out), lambda i: (i, 0)),
        out_shape=jax.ShapeDtypeStruct((n, d_out), jnp.float32),
    )(acc2, acc2, p16, w2, b2row)


def _make_sc_aggregate(n, e):
    info = plsc.get_sparse_core_info()
    nc, ns, lanes = info.num_cores, info.num_subcores, info.num_lanes
    assert lanes == _LANES and n % ns == 0 and e % nc == 0
    npw = n // ns
    half = e // nc
    ch = 6400
    unroll = 4
    assert half % ch == 0 and (ch // _LANES) % unroll == 0
    nchunk = half // ch
    gb = 512

    mesh = plsc.VectorSubcoreMesh(core_axis_name="c", subcore_axis_name="s")

    @functools.partial(
        pl.kernel,
        mesh=mesh,
        compiler_params=pltpu.CompilerParams(
            needs_layout_passes=False, use_tc_tiling_on_sc=False
        ),
        out_type=jax.ShapeDtypeStruct((nc * n * _LANES,), jnp.float32),
        scratch_types=[
            pltpu.VMEM((npw * _LANES,), jnp.float32),
            pltpu.VMEM((ch,), jnp.int32),
            pltpu.VMEM((ch + _LANES,), jnp.int32),
            pltpu.VMEM((gb, _LANES), jnp.float32),
            pltpu.VMEM((gb, _LANES), jnp.float32),
            pltpu.VMEM((gb,), jnp.int32),
            pltpu.VMEM((gb,), jnp.int32),
            pltpu.SemaphoreType.DMA,
        ],
    )
    def sc_agg(dst_hbm, rec_hbm, q_hbm, acc_hbm,
               acc_v, dst_v, cid_v, rec_v, qrow_v, srcb_v, dstl_v, sem):
        cid = lax.axis_index("c")
        sid = lax.axis_index("s")
        node_lo = sid * npw
        iota = lax.iota(jnp.int32, _LANES)
        ninf = jnp.full((_LANES,), -jnp.inf, jnp.float32)

        def init_body(r, _):
            plsc.store_scatter(acc_v, [r * _LANES + iota], ninf)
            return 0
        lax.fori_loop(0, npw, init_body, 0)

        def cinit_body(r, _):
            plsc.store_scatter(cid_v, [r * _LANES + iota],
                               jnp.full((_LANES,), sid, jnp.int32))
            return 0
        lax.fori_loop(0, (ch + _LANES) // _LANES, cinit_body, 0)

        def accum_batch(b, k):
            pltpu.async_copy(
                rec_hbm.at[cid_v.at[pl.ds(b * gb, gb)]], rec_v, sem
            ).wait()
            valid = k - b * gb
            for g in range(gb // _LANES):
                rows = g * _LANES + iota
                srcf = plsc.load_gather(
                    rec_v, [rows, jnp.full((_LANES,), 10, jnp.int32)])
                srci = plsc.bitcast(srcf, jnp.int32)
                srci = jnp.where(rows < valid, srci,
                                 jnp.full((_LANES,), node_lo, jnp.int32))
                srcb_v[g * _LANES:(g + 1) * _LANES] = srci
                dstf = plsc.load_gather(
                    rec_v, [rows, jnp.full((_LANES,), 11, jnp.int32)])
                dsti = plsc.bitcast(dstf, jnp.int32) - node_lo
                dsti = jnp.where(rows < valid, dsti,
                                 jnp.zeros((_LANES,), jnp.int32))
                dstl_v[g * _LANES:(g + 1) * _LANES] = dsti * _LANES
            pltpu.async_copy(q_hbm.at[srcb_v], qrow_v, sem).wait()

            def one_edge(j):
                rows = jnp.full((_LANES,), j, jnp.int32)
                dvec = plsc.load_gather(dstl_v, [rows])
                rrow = plsc.load_gather(rec_v, [rows, iota])
                qrow = plsc.load_gather(qrow_v, [rows, iota])
                aidx = dvec + iota
                arow = plsc.load_gather(acc_v, [aidx])
                plsc.store_scatter(acc_v, [aidx],
                                   jnp.maximum(arow, rrow + qrow))

            def edge_body4(i, _):
                for u in range(4):
                    one_edge(i * 4 + u)
                return 0

            def edge_body1(j, _):
                one_edge(j)
                return 0
            nvalid = jnp.minimum(valid, gb)
            n4 = nvalid // 4
            lax.fori_loop(0, n4, edge_body4, 0)
            lax.fori_loop(n4 * 4, nvalid, edge_body1, 0)
            return k

        def chunk_body(c, _):
            base = cid * half + c * ch
            pltpu.sync_copy(dst_hbm.at[pl.ds(base, ch)], dst_v)

            def scan_body(t, cur):
                i0 = t * unroll
                masks, cnts, idsl = [], [], []
                for u in range(unroll):
                    dvec = plsc.load_gather(dst_v, [(i0 + u) * _LANES + iota])
                    du = dvec - node_lo
                    m = (du >= 0) & (du < npw)
                    masks.append(m)
                    idsl.append(base + (i0 + u) * _LANES + iota)
                    cnts.append(jnp.sum(m.astype(jnp.int32)))
                cc = cur
                for u in range(unroll):
                    plsc.store_compressed(
                        cid_v.at[pl.ds(cc, _LANES)], idsl[u], mask=masks[u])
                    cc = cc + cnts[u]
                return cc

            k = lax.fori_loop(0, ch // _LANES // unroll, scan_body, 0)
            nb = (k + gb - 1) // gb
            lax.fori_loop(0, nb, accum_batch, k)
            return 0

        lax.fori_loop(0, nchunk, chunk_body, 0)

        out_off = (cid * n + node_lo) * _LANES
        pltpu.sync_copy(acc_v, acc_hbm.at[pl.ds(out_off, npw * _LANES)])

    return sc_agg


def kernel(x, edge_index, dist, W1, b1, W2, b2):
    n, d_feat = x.shape
    e = edge_index.shape[1]
    d_hid = W1.shape[1]
    src = edge_index[0]
    dst = edge_index[1]
    A = W1[0:d_feat]
    B = W1[d_feat:2 * d_feat]
    C = W1[2 * d_feat:]

    zpad = jnp.zeros((d_feat, _LANES - d_hid), jnp.float32)
    w_big = jnp.concatenate([A - B, zpad, B, zpad], axis=1)
    p16, q16 = _node_prep(x, w_big, block=4000)

    c16 = jnp.concatenate([C, jnp.zeros((2, _LANES - d_hid), jnp.float32)], axis=1)
    b16 = jnp.concatenate([b1, jnp.zeros((_LANES - d_hid,), jnp.float32)]).reshape(1, _LANES)
    srcf = lax.bitcast_convert_type(src, jnp.float32).reshape(e, 1)
    dstf = lax.bitcast_convert_type(dst, jnp.float32).reshape(e, 1)
    rec = _edge_prep(dist, srcf, dstf, c16, b16, block=8000)

    accf = _make_sc_aggregate(n, e)(dst, rec, q16)
    acc2 = accf.reshape(2 * n, _LANES)

    return _epilogue(acc2, p16, W2, b2.reshape(1, -1), block=4000)


# speedup vs baseline: 1.1699x; 1.0271x over previous
import functools

import jax
import jax.numpy as jnp
from jax import lax
from jax.experimental import pallas as pl
from jax.experimental.pallas import tpu as pltpu
from jax.experimental.pallas import tpu_sc as plsc

_LANES = 16


def _node_prep_body(x_ref, w_ref, p_ref, q_ref):
    xw = jnp.dot(x_ref[...], w_ref[...], preferred_element_type=jnp.float32)
    p_ref[...] = xw[:, :_LANES]
    q_ref[...] = xw[:, _LANES:]


def _node_prep(x, w_big, block):
    n, d = x.shape
    return pl.pallas_call(
        _node_prep_body,
        grid=(n // block,),
        in_specs=[
            pl.BlockSpec((block, d), lambda i: (i, 0)),
            pl.BlockSpec(w_big.shape, lambda i: (0, 0)),
        ],
        out_specs=[
            pl.BlockSpec((block, _LANES), lambda i: (i, 0)),
            pl.BlockSpec((block, _LANES), lambda i: (i, 0)),
        ],
        out_shape=[
            jax.ShapeDtypeStruct((n, _LANES), jnp.float32),
            jax.ShapeDtypeStruct((n, _LANES), jnp.float32),
        ],
    )(x, w_big)


def _edge_prep_body(dist_ref, srcf_ref, dstf_ref, c_ref, b_ref, rec_ref):
    r = jnp.dot(dist_ref[...], c_ref[...], preferred_element_type=jnp.float32)
    r = r + b_ref[...]
    rec_ref[...] = jnp.concatenate(
        [
            r[:, :10],
            srcf_ref[...],
            dstf_ref[...],
            jnp.zeros((r.shape[0], 4), jnp.float32),
        ],
        axis=1,
    )


def _edge_prep(dist, srcf, dstf, c16, b16, block):
    e = dist.shape[0]
    return pl.pallas_call(
        _edge_prep_body,
        grid=(e // block,),
        in_specs=[
            pl.BlockSpec((block, 2), lambda i: (i, 0)),
            pl.BlockSpec((block, 1), lambda i: (i, 0)),
            pl.BlockSpec((block, 1), lambda i: (i, 0)),
            pl.BlockSpec(c16.shape, lambda i: (0, 0)),
            pl.BlockSpec(b16.shape, lambda i: (0, 0)),
        ],
        out_specs=pl.BlockSpec((block, _LANES), lambda i: (i, 0)),
        out_shape=jax.ShapeDtypeStruct((e, _LANES), jnp.float32),
    )(dist, srcf, dstf, c16, b16)


def _epilogue_body(a0_ref, a1_ref, p_ref, w2_ref, b2_ref, o_ref):
    a = jnp.maximum(a0_ref[...][:, :10], a1_ref[...][:, :10])
    p = p_ref[...][:, :10]
    m = jnp.where(jnp.isneginf(a), 0.0, a + p)
    o_ref[...] = jnp.dot(m, w2_ref[...], preferred_element_type=jnp.float32) + b2_ref[...]


def _epilogue(acc2, p16, w2, b2row, block):
    n = p16.shape[0]
    nblk = n // block
    d_out = w2.shape[1]
    return pl.pallas_call(
        _epilogue_body,
        grid=(nblk,),
        in_specs=[
            pl.BlockSpec((block, _LANES), lambda i: (i, 0)),
            pl.BlockSpec((block, _LANES), lambda i: (i + nblk, 0)),
            pl.BlockSpec((block, _LANES), lambda i: (i, 0)),
            pl.BlockSpec(w2.shape, lambda i: (0, 0)),
            pl.BlockSpec(b2row.shape, lambda i: (0, 0)),
        ],
        out_specs=pl.BlockSpec((block, d_out), lambda i: (i, 0)),
        out_shape=jax.ShapeDtypeStruct((n, d_out), jnp.float32),
    )(acc2, acc2, p16, w2, b2row)


def _make_sc_aggregate(n, e):
    info = plsc.get_sparse_core_info()
    nc, ns, lanes = info.num_cores, info.num_subcores, info.num_lanes
    assert lanes == _LANES and n % ns == 0 and e % nc == 0
    npw = n // ns
    half = e // nc
    ch = 6400
    unroll = 4
    assert half % ch == 0 and (ch // _LANES) % unroll == 0
    nchunk = half // ch
    gb = 512

    mesh = plsc.VectorSubcoreMesh(core_axis_name="c", subcore_axis_name="s")

    @functools.partial(
        pl.kernel,
        mesh=mesh,
        compiler_params=pltpu.CompilerParams(
            needs_layout_passes=False, use_tc_tiling_on_sc=False
        ),
        out_type=jax.ShapeDtypeStruct((nc * n * _LANES,), jnp.float32),
        scratch_types=[
            pltpu.VMEM((npw * _LANES,), jnp.float32),
            pltpu.VMEM((ch,), jnp.int32),
            pltpu.VMEM((ch + _LANES,), jnp.int32),
            pltpu.VMEM((gb, _LANES), jnp.float32),
            pltpu.VMEM((gb, _LANES), jnp.float32),
            pltpu.VMEM((gb,), jnp.int32),
            pltpu.VMEM((gb,), jnp.int32),
            pltpu.SemaphoreType.DMA,
        ],
    )
    def sc_agg(dst_hbm, rec_hbm, q_hbm, acc_hbm,
               acc_v, dst_v, cid_v, rec_v, qrow_v, srcb_v, dstl_v, sem):
        cid = lax.axis_index("c")
        sid = lax.axis_index("s")
        node_lo = sid * npw
        iota = lax.iota(jnp.int32, _LANES)
        ninf = jnp.full((_LANES,), -jnp.inf, jnp.float32)

        def init_body(r, _):
            plsc.store_scatter(acc_v, [r * _LANES + iota], ninf)
            return 0
        lax.fori_loop(0, npw, init_body, 0)

        def cinit_body(r, _):
            plsc.store_scatter(cid_v, [r * _LANES + iota],
                               jnp.full((_LANES,), sid, jnp.int32))
            return 0
        lax.fori_loop(0, (ch + _LANES) // _LANES, cinit_body, 0)

        def accum_batch(b, k):
            pltpu.async_copy(
                rec_hbm.at[cid_v.at[pl.ds(b * gb, gb)]], rec_v, sem
            ).wait()
            valid = k - b * gb
            for g in range(gb // _LANES):
                rows = g * _LANES + iota
                srcf = plsc.load_gather(
                    rec_v, [rows, jnp.full((_LANES,), 10, jnp.int32)])
                srci = plsc.bitcast(srcf, jnp.int32)
                srci = jnp.where(rows < valid, srci,
                                 jnp.full((_LANES,), node_lo, jnp.int32))
                srcb_v[g * _LANES:(g + 1) * _LANES] = srci
                dstf = plsc.load_gather(
                    rec_v, [rows, jnp.full((_LANES,), 11, jnp.int32)])
                dsti = plsc.bitcast(dstf, jnp.int32) - node_lo
                dsti = jnp.where(rows < valid, dsti,
                                 jnp.zeros((_LANES,), jnp.int32))
                dstl_v[g * _LANES:(g + 1) * _LANES] = dsti * _LANES
            pltpu.async_copy(q_hbm.at[srcb_v], qrow_v, sem).wait()

            def one_edge(j):
                rows = jnp.full((_LANES,), j, jnp.int32)
                dvec = plsc.load_gather(dstl_v, [rows])
                rrow = plsc.load_gather(rec_v, [rows, iota])
                qrow = plsc.load_gather(qrow_v, [rows, iota])
                aidx = dvec + iota
                arow = plsc.load_gather(acc_v, [aidx])
                plsc.store_scatter(acc_v, [aidx],
                                   jnp.maximum(arow, rrow + qrow))

            def edge_body1(j, _):
                one_edge(j)
                return 0

            grp = 8

            def group_body(g, _):
                j0 = g * grp
                abase = plsc.load_gather(dstl_v, [j0 + iota])
                abase = jnp.where(iota < grp, abase, -(iota + 1))
                cnts, _lm = plsc.scan_count(abase)
                has_dup = jnp.max(cnts) != jnp.min(cnts)

                def slow(_):
                    lax.fori_loop(j0, j0 + grp, edge_body1, 0)
                    return 0

                def fast(_):
                    aidxs, news = [], []
                    for u in range(grp):
                        rw = jnp.full((_LANES,), j0 + u, jnp.int32)
                        dv = plsc.load_gather(dstl_v, [rw])
                        aidx = dv + iota
                        rrow = plsc.load_gather(rec_v, [rw, iota])
                        qrow = plsc.load_gather(qrow_v, [rw, iota])
                        arow = plsc.load_gather(acc_v, [aidx])
                        aidxs.append(aidx)
                        news.append(jnp.maximum(arow, rrow + qrow))
                    for u in range(grp):
                        plsc.store_scatter(acc_v, [aidxs[u]], news[u])
                    return 0

                lax.cond(has_dup, slow, fast, 0)
                return 0

            nvalid = jnp.minimum(valid, gb)
            ng = nvalid // grp
            lax.fori_loop(0, ng, group_body, 0)
            lax.fori_loop(ng * grp, nvalid, edge_body1, 0)
            return k

        def chunk_body(c, _):
            base = cid * half + c * ch
            pltpu.sync_copy(dst_hbm.at[pl.ds(base, ch)], dst_v)

            def scan_body(t, cur):
                i0 = t * unroll
                masks, cnts, idsl = [], [], []
                for u in range(unroll):
                    dvec = plsc.load_gather(dst_v, [(i0 + u) * _LANES + iota])
                    du = dvec - node_lo
                    m = (du >= 0) & (du < npw)
                    masks.append(m)
                    idsl.append(base + (i0 + u) * _LANES + iota)
                    cnts.append(jnp.sum(m.astype(jnp.int32)))
                cc = cur
                for u in range(unroll):
                    plsc.store_compressed(
                        cid_v.at[pl.ds(cc, _LANES)], idsl[u], mask=masks[u])
                    cc = cc + cnts[u]
                return cc

            k = lax.fori_loop(0, ch // _LANES // unroll, scan_body, 0)
            nb = (k + gb - 1) // gb
            lax.fori_loop(0, nb, accum_batch, k)
            return 0

        lax.fori_loop(0, nchunk, chunk_body, 0)

        out_off = (cid * n + node_lo) * _LANES
        pltpu.sync_copy(acc_v, acc_hbm.at[pl.ds(out_off, npw * _LANES)])

    return sc_agg


def kernel(x, edge_index, dist, W1, b1, W2, b2):
    n, d_feat = x.shape
    e = edge_index.shape[1]
    d_hid = W1.shape[1]
    src = edge_index[0]
    dst = edge_index[1]
    A = W1[0:d_feat]
    B = W1[d_feat:2 * d_feat]
    C = W1[2 * d_feat:]

    zpad = jnp.zeros((d_feat, _LANES - d_hid), jnp.float32)
    w_big = jnp.concatenate([A - B, zpad, B, zpad], axis=1)
    p16, q16 = _node_prep(x, w_big, block=4000)

    c16 = jnp.concatenate([C, jnp.zeros((2, _LANES - d_hid), jnp.float32)], axis=1)
    b16 = jnp.concatenate([b1, jnp.zeros((_LANES - d_hid,), jnp.float32)]).reshape(1, _LANES)
    srcf = lax.bitcast_convert_type(src, jnp.float32).reshape(e, 1)
    dstf = lax.bitcast_convert_type(dst, jnp.float32).reshape(e, 1)
    rec = _edge_prep(dist, srcf, dstf, c16, b16, block=8000)

    accf = _make_sc_aggregate(n, e)(dst, rec, q16)
    acc2 = accf.reshape(2 * n, _LANES)

    return _epilogue(acc2, p16, W2, b2.reshape(1, -1), block=4000)


# speedup vs baseline: 1.2049x; 1.0299x over previous
import functools

import jax
import jax.numpy as jnp
from jax import lax
from jax.experimental import pallas as pl
from jax.experimental.pallas import tpu as pltpu
from jax.experimental.pallas import tpu_sc as plsc

_LANES = 16


def _node_prep_body(x_ref, w_ref, p_ref, q_ref):
    xw = jnp.dot(x_ref[...], w_ref[...], preferred_element_type=jnp.float32)
    p_ref[...] = xw[:, :_LANES]
    q_ref[...] = xw[:, _LANES:]


def _node_prep(x, w_big, block):
    n, d = x.shape
    return pl.pallas_call(
        _node_prep_body,
        grid=(n // block,),
        in_specs=[
            pl.BlockSpec((block, d), lambda i: (i, 0)),
            pl.BlockSpec(w_big.shape, lambda i: (0, 0)),
        ],
        out_specs=[
            pl.BlockSpec((block, _LANES), lambda i: (i, 0)),
            pl.BlockSpec((block, _LANES), lambda i: (i, 0)),
        ],
        out_shape=[
            jax.ShapeDtypeStruct((n, _LANES), jnp.float32),
            jax.ShapeDtypeStruct((n, _LANES), jnp.float32),
        ],
    )(x, w_big)


def _edge_prep_body(dist_ref, srcf_ref, dstf_ref, c_ref, b_ref, rec_ref):
    r = jnp.dot(dist_ref[...], c_ref[...], preferred_element_type=jnp.float32)
    r = r + b_ref[...]
    rec_ref[...] = jnp.concatenate(
        [
            r[:, :10],
            srcf_ref[...],
            dstf_ref[...],
            jnp.zeros((r.shape[0], 4), jnp.float32),
        ],
        axis=1,
    )


def _edge_prep(dist, srcf, dstf, c16, b16, block):
    e = dist.shape[0]
    return pl.pallas_call(
        _edge_prep_body,
        grid=(e // block,),
        in_specs=[
            pl.BlockSpec((block, 2), lambda i: (i, 0)),
            pl.BlockSpec((block, 1), lambda i: (i, 0)),
            pl.BlockSpec((block, 1), lambda i: (i, 0)),
            pl.BlockSpec(c16.shape, lambda i: (0, 0)),
            pl.BlockSpec(b16.shape, lambda i: (0, 0)),
        ],
        out_specs=pl.BlockSpec((block, _LANES), lambda i: (i, 0)),
        out_shape=jax.ShapeDtypeStruct((e, _LANES), jnp.float32),
    )(dist, srcf, dstf, c16, b16)


def _epilogue_body(a0_ref, a1_ref, p_ref, w2_ref, b2_ref, o_ref):
    a = jnp.maximum(a0_ref[...][:, :10], a1_ref[...][:, :10])
    p = p_ref[...][:, :10]
    m = jnp.where(jnp.isneginf(a), 0.0, a + p)
    o_ref[...] = jnp.dot(m, w2_ref[...], preferred_element_type=jnp.float32) + b2_ref[...]


def _epilogue(acc2, p16, w2, b2row, block):
    n = p16.shape[0]
    nblk = n // block
    d_out = w2.shape[1]
    return pl.pallas_call(
        _epilogue_body,
        grid=(nblk,),
        in_specs=[
            pl.BlockSpec((block, _LANES), lambda i: (i, 0)),
            pl.BlockSpec((block, _LANES), lambda i: (i + nblk, 0)),
            pl.BlockSpec((block, _LANES), lambda i: (i, 0)),
            pl.BlockSpec(w2.shape, lambda i: (0, 0)),
            pl.BlockSpec(b2row.shape, lambda i: (0, 0)),
        ],
        out_specs=pl.BlockSpec((block, d_out), lambda i: (i, 0)),
        out_shape=jax.ShapeDtypeStruct((n, d_out), jnp.float32),
    )(acc2, acc2, p16, w2, b2row)


def _make_sc_aggregate(n, e):
    info = plsc.get_sparse_core_info()
    nc, ns, lanes = info.num_cores, info.num_subcores, info.num_lanes
    assert lanes == _LANES and n % ns == 0 and e % nc == 0
    npw = n // ns
    half = e // nc
    ch = 6400
    unroll = 8
    assert half % ch == 0 and (ch // _LANES) % unroll == 0
    nchunk = half // ch
    gb = 512

    mesh = plsc.VectorSubcoreMesh(core_axis_name="c", subcore_axis_name="s")

    @functools.partial(
        pl.kernel,
        mesh=mesh,
        compiler_params=pltpu.CompilerParams(
            needs_layout_passes=False, use_tc_tiling_on_sc=False
        ),
        out_type=jax.ShapeDtypeStruct((nc * n * _LANES,), jnp.float32),
        scratch_types=[
            pltpu.VMEM((npw * _LANES,), jnp.float32),
            pltpu.VMEM((ch,), jnp.int32),
            pltpu.VMEM((ch + _LANES,), jnp.int32),
            pltpu.VMEM((gb, _LANES), jnp.float32),
            pltpu.VMEM((gb, _LANES), jnp.float32),
            pltpu.VMEM((gb,), jnp.int32),
            pltpu.VMEM((gb,), jnp.int32),
            pltpu.SemaphoreType.DMA,
        ],
    )
    def sc_agg(dst_hbm, rec_hbm, q_hbm, acc_hbm,
               acc_v, dst_v, cid_v, rec_v, qrow_v, srcb_v, dstl_v, sem):
        cid = lax.axis_index("c")
        sid = lax.axis_index("s")
        node_lo = sid * npw
        iota = lax.iota(jnp.int32, _LANES)
        ninf = jnp.full((_LANES,), -jnp.inf, jnp.float32)

        def init_body(r, _):
            plsc.store_scatter(acc_v, [r * _LANES + iota], ninf)
            return 0
        lax.fori_loop(0, npw, init_body, 0)

        def cinit_body(r, _):
            plsc.store_scatter(cid_v, [r * _LANES + iota],
                               jnp.full((_LANES,), sid, jnp.int32))
            return 0
        lax.fori_loop(0, (ch + _LANES) // _LANES, cinit_body, 0)

        def accum_batch(b, k):
            pltpu.async_copy(
                rec_hbm.at[cid_v.at[pl.ds(b * gb, gb)]], rec_v, sem
            ).wait()
            valid = k - b * gb
            for g in range(gb // _LANES):
                rows = g * _LANES + iota
                srcf = plsc.load_gather(
                    rec_v, [rows, jnp.full((_LANES,), 10, jnp.int32)])
                srci = plsc.bitcast(srcf, jnp.int32)
                srci = jnp.where(rows < valid, srci,
                                 jnp.full((_LANES,), node_lo, jnp.int32))
                srcb_v[g * _LANES:(g + 1) * _LANES] = srci
                dstf = plsc.load_gather(
                    rec_v, [rows, jnp.full((_LANES,), 11, jnp.int32)])
                dsti = plsc.bitcast(dstf, jnp.int32) - node_lo
                dsti = jnp.where(rows < valid, dsti,
                                 jnp.zeros((_LANES,), jnp.int32))
                dstl_v[g * _LANES:(g + 1) * _LANES] = dsti * _LANES
            pltpu.async_copy(q_hbm.at[srcb_v], qrow_v, sem).wait()

            def one_edge(j):
                rows = jnp.full((_LANES,), j, jnp.int32)
                dvec = plsc.load_gather(dstl_v, [rows])
                rrow = plsc.load_gather(rec_v, [rows, iota])
                qrow = plsc.load_gather(qrow_v, [rows, iota])
                aidx = dvec + iota
                arow = plsc.load_gather(acc_v, [aidx])
                plsc.store_scatter(acc_v, [aidx],
                                   jnp.maximum(arow, rrow + qrow))

            def edge_body1(j, _):
                one_edge(j)
                return 0

            grp = 8

            def group_body(g, _):
                j0 = g * grp
                abase = plsc.load_gather(dstl_v, [j0 + iota])
                abase = jnp.where(iota < grp, abase, -(iota + 1))
                cnts, _lm = plsc.scan_count(abase)
                has_dup = jnp.max(cnts) != jnp.min(cnts)

                def slow(_):
                    lax.fori_loop(j0, j0 + grp, edge_body1, 0)
                    return 0

                def fast(_):
                    aidxs, news = [], []
                    for u in range(grp):
                        rw = jnp.full((_LANES,), j0 + u, jnp.int32)
                        dv = plsc.load_gather(dstl_v, [rw])
                        aidx = dv + iota
                        rrow = plsc.load_gather(rec_v, [rw, iota])
                        qrow = plsc.load_gather(qrow_v, [rw, iota])
                        arow = plsc.load_gather(acc_v, [aidx])
                        aidxs.append(aidx)
                        news.append(jnp.maximum(arow, rrow + qrow))
                    for u in range(grp):
                        plsc.store_scatter(acc_v, [aidxs[u]], news[u])
                    return 0

                lax.cond(has_dup, slow, fast, 0)
                return 0

            nvalid = jnp.minimum(valid, gb)
            ng = nvalid // grp
            lax.fori_loop(0, ng, group_body, 0)
            lax.fori_loop(ng * grp, nvalid, edge_body1, 0)
            return k

        def chunk_body(c, _):
            base = cid * half + c * ch
            pltpu.sync_copy(dst_hbm.at[pl.ds(base, ch)], dst_v)

            def scan_body(t, cur):
                i0 = t * unroll
                masks, cnts, idsl = [], [], []
                for u in range(unroll):
                    dvec = plsc.load_gather(dst_v, [(i0 + u) * _LANES + iota])
                    du = dvec - node_lo
                    m = (du >= 0) & (du < npw)
                    masks.append(m)
                    idsl.append(base + (i0 + u) * _LANES + iota)
                    cnts.append(jnp.sum(m.astype(jnp.int32)))
                cc = cur
                for u in range(unroll):
                    plsc.store_compressed(
                        cid_v.at[pl.ds(cc, _LANES)], idsl[u], mask=masks[u])
                    cc = cc + cnts[u]
                return cc

            k = lax.fori_loop(0, ch // _LANES // unroll, scan_body, 0)
            nb = (k + gb - 1) // gb
            lax.fori_loop(0, nb, accum_batch, k)
            return 0

        lax.fori_loop(0, nchunk, chunk_body, 0)

        out_off = (cid * n + node_lo) * _LANES
        pltpu.sync_copy(acc_v, acc_hbm.at[pl.ds(out_off, npw * _LANES)])

    return sc_agg


def kernel(x, edge_index, dist, W1, b1, W2, b2):
    n, d_feat = x.shape
    e = edge_index.shape[1]
    d_hid = W1.shape[1]
    src = edge_index[0]
    dst = edge_index[1]
    A = W1[0:d_feat]
    B = W1[d_feat:2 * d_feat]
    C = W1[2 * d_feat:]

    zpad = jnp.zeros((d_feat, _LANES - d_hid), jnp.float32)
    w_big = jnp.concatenate([A - B, zpad, B, zpad], axis=1)
    p16, q16 = _node_prep(x, w_big, block=4000)

    c16 = jnp.concatenate([C, jnp.zeros((2, _LANES - d_hid), jnp.float32)], axis=1)
    b16 = jnp.concatenate([b1, jnp.zeros((_LANES - d_hid,), jnp.float32)]).reshape(1, _LANES)
    srcf = lax.bitcast_convert_type(src, jnp.float32).reshape(e, 1)
    dstf = lax.bitcast_convert_type(dst, jnp.float32).reshape(e, 1)
    rec = _edge_prep(dist, srcf, dstf, c16, b16, block=8000)

    accf = _make_sc_aggregate(n, e)(dst, rec, q16)
    acc2 = accf.reshape(2 * n, _LANES)

    return _epilogue(acc2, p16, W2, b2.reshape(1, -1), block=4000)
